# baseline (speedup 1.0000x reference)
import dataclasses
import functools

import jax
import jax.numpy as jnp
from jax import lax
from jax.experimental import pallas as pl
from jax.experimental.pallas import tpu as pltpu
from jax.experimental.pallas import tpu_sc as plsc

N = 16384
NNZ = 268435
D = 256

NC = 2
NS = 16
NW = NC * NS
L = 16

RPL = 16
ROWS_PER_GROUP = RPL * L
NUM_GROUPS = N // ROWS_PER_GROUP
GROUPS_PER_TILE = NUM_GROUPS // NW
NUM_BOUND = N // RPL + 1
OFFS_PAD = 1056
WBUF = 4096
NBUF = 4
NNZ_PAD = ((NNZ + WBUF + 7) // 8) * 8


def _sc_spmm(rows_p, cols_p, vals_p, offs, weight):
    mesh = plsc.VectorSubcoreMesh(core_axis_name="c", subcore_axis_name="s")
    cp = pltpu.CompilerParams()
    if "needs_layout_passes" in pltpu.CompilerParams.__dataclass_fields__:
        cp = dataclasses.replace(cp, needs_layout_passes=False)

    @functools.partial(
        pl.kernel,
        compiler_params=cp,
        out_type=jax.ShapeDtypeStruct((N * D,), jnp.float32),
        mesh=mesh,
        scratch_types=[
            pltpu.VMEM((OFFS_PAD,), jnp.int32),
            pltpu.VMEM((WBUF,), jnp.int32),
            pltpu.VMEM((WBUF,), jnp.int32),
            pltpu.VMEM((WBUF,), jnp.float32),
            [pltpu.VMEM((L,), jnp.int32) for _ in range(NBUF)],
            [pltpu.VMEM((L, D), jnp.float32) for _ in range(NBUF)],
            pltpu.VMEM((ROWS_PER_GROUP * D,), jnp.float32),
            [pltpu.SemaphoreType.DMA for _ in range(NBUF)],
        ],
    )
    def sc_kernel(rows_hbm, cols_hbm, vals_hbm, offs_hbm, w_hbm, out_hbm,
                  offs_v, rows_buf, cols_buf, vals_buf, cidx, g, acc_v, sems):
        wid = lax.axis_index("s") * NC + lax.axis_index("c")
        pltpu.sync_copy(offs_hbm, offs_v)
        lane = lax.broadcasted_iota(jnp.int32, (L,), 0)
        zero16 = jnp.zeros((L,), jnp.float32)

        for cc in range(GROUPS_PER_TILE):
            c = wid * GROUPS_PER_TILE + cc
            base_row = c * ROWS_PER_GROUP
            starts = offs_v[pl.ds(c * L, L)]
            ends = offs_v[pl.ds(c * L + 1, L)]

            @pl.loop(0, ROWS_PER_GROUP * D // L, unroll=8)
            def _(i):
                acc_v[pl.ds(i * L, L)] = zero16

            start_g = jnp.min(starts)
            end_g = jnp.max(ends)
            a0 = start_g - (start_g & 7)
            nsw = (end_g - a0 + WBUF - 1) // WBUF

            @pl.loop(0, nsw)
            def _(w):
                k0g = pl.multiple_of(a0 + w * WBUF, 8)
                pltpu.sync_copy(rows_hbm.at[pl.ds(k0g, WBUF)], rows_buf)
                pltpu.sync_copy(cols_hbm.at[pl.ds(k0g, WBUF)], cols_buf)
                pltpu.sync_copy(vals_hbm.at[pl.ds(k0g, WBUF)], vals_buf)
                s_rel = jnp.clip(starts - k0g, 0, WBUF)
                e_rel = jnp.clip(ends - k0g, 0, WBUF)
                cnt = jnp.maximum(e_rel - s_rel, 0)
                cnt_m1 = jnp.maximum(cnt - 1, 0)
                steps = jnp.max(cnt)
                ngrp = (steps + NBUF - 1) // NBUF

                def krel(t):
                    return jnp.minimum(s_rel + jnp.minimum(t, cnt_m1),
                                       WBUF - 1)

                def issue(t, b):
                    cidx[b][...] = plsc.load_gather(cols_buf, [krel(t)])
                    pltpu.async_copy(w_hbm.at[cidx[b]], g[b], sems[b])

                def compute(t, b):
                    k_rel = krel(t)
                    valid = t < cnt
                    vals16 = plsc.load_gather(vals_buf, [k_rel])
                    rows16 = plsc.load_gather(rows_buf, [k_rel])
                    v_eff = jnp.where(valid, vals16, 0.0)
                    lr = jnp.clip(rows16 - base_row, 0, ROWS_PER_GROUP - 1)
                    abase = lr * D

                    @plsc.parallel_loop(0, D, unroll=8)
                    def _(d):
                        dfull = jnp.zeros((L,), jnp.int32) + d
                        g16 = plsc.load_gather(g[b], [lane, dfull])
                        plsc.addupdate_scatter(acc_v, [abase + d],
                                               v_eff * g16)

                for b in range(NBUF):
                    issue(b, b)

                @pl.loop(0, ngrp)
                def _(u):
                    for b in range(NBUF):
                        t = u * NBUF + b
                        pltpu.make_async_copy(
                            w_hbm.at[cidx[b]], g[b], sems[b]).wait()
                        compute(t, b)
                        issue(t + NBUF, b)

                for b in range(NBUF):
                    pltpu.make_async_copy(
                        w_hbm.at[cidx[b]], g[b], sems[b]).wait()

            pltpu.sync_copy(
                acc_v, out_hbm.at[pl.ds(base_row * D, ROWS_PER_GROUP * D)])

    return sc_kernel(rows_p, cols_p, vals_p, offs, weight)


def kernel(inp_rows, inp_cols, inp_values, weight):
    offs = jnp.searchsorted(
        inp_rows, jnp.arange(0, N + 1, RPL), side="left"
    ).astype(jnp.int32)
    offs = jnp.pad(offs, (0, OFFS_PAD - offs.shape[0]), constant_values=NNZ)
    pad = NNZ_PAD - NNZ
    rows_p = jnp.pad(inp_rows, (0, pad), constant_values=N - 1)
    cols_p = jnp.pad(inp_cols, (0, pad), constant_values=0)
    vals_p = jnp.pad(inp_values, (0, pad), constant_values=0.0)
    out_flat = _sc_spmm(rows_p, cols_p, vals_p, offs, weight)
    return out_flat.reshape(N, D)


# speedup vs baseline: 9.7290x; 9.7290x over previous
import dataclasses
import functools

import jax
import jax.numpy as jnp
from jax import lax
from jax.experimental import pallas as pl
from jax.experimental.pallas import tpu as pltpu
from jax.experimental.pallas import tpu_sc as plsc

N = 16384
NNZ = 268435
D = 256

NC = 2
NS = 16
NW = NC * NS
L = 16

ROWS_PER_GROUP = 256
NUM_GROUPS = N // ROWS_PER_GROUP
GROUPS_PER_TILE = NUM_GROUPS // NW
OFFS_PAD = 80
WBUF = 4096
W = 32
NBUF = 4
NNZ_PAD = ((NNZ + WBUF + 7) // 8) * 8

_GATHER_DNUMS = lax.GatherDimensionNumbers(
    offset_dims=(), collapsed_slice_dims=(0,), start_index_map=(0,))


def _bcast_lane(v, idx):
    return lax.gather(v, idx[:, None], _GATHER_DNUMS, (1,),
                      mode=lax.GatherScatterMode.PROMISE_IN_BOUNDS)


def _sc_spmm(rows_p, cols_p, vals_p, offs, weight):
    mesh = plsc.VectorSubcoreMesh(core_axis_name="c", subcore_axis_name="s")
    cp = pltpu.CompilerParams()
    if "needs_layout_passes" in pltpu.CompilerParams.__dataclass_fields__:
        cp = dataclasses.replace(cp, needs_layout_passes=False)

    @functools.partial(
        pl.kernel,
        compiler_params=cp,
        out_type=jax.ShapeDtypeStruct((N * D,), jnp.float32),
        mesh=mesh,
        scratch_types=[
            pltpu.VMEM((OFFS_PAD,), jnp.int32),
            pltpu.VMEM((WBUF,), jnp.int32),
            pltpu.VMEM((WBUF,), jnp.int32),
            pltpu.VMEM((WBUF,), jnp.float32),
            [pltpu.VMEM((W, D), jnp.float32) for _ in range(NBUF)],
            pltpu.VMEM((ROWS_PER_GROUP * D,), jnp.float32),
            [pltpu.SemaphoreType.DMA for _ in range(NBUF)],
        ],
    )
    def sc_kernel(rows_hbm, cols_hbm, vals_hbm, offs_hbm, w_hbm, out_hbm,
                  offs_v, rows_buf, cols_buf, vals_buf, g, acc_v, sems):
        wid = lax.axis_index("s") * NC + lax.axis_index("c")
        pltpu.sync_copy(offs_hbm, offs_v)
        lane = lax.broadcasted_iota(jnp.int32, (L,), 0)
        zero16 = jnp.zeros((L,), jnp.float32)
        idx1 = [dj * L + lane for dj in range(D // L)]

        for cc in range(GROUPS_PER_TILE):
            c = wid * GROUPS_PER_TILE + cc
            base_row = c * ROWS_PER_GROUP
            ov = offs_v[pl.ds(c, L)]
            start = jnp.sum(jnp.where(lane == 0, ov, 0))
            end = jnp.sum(jnp.where(lane == 1, ov, 0))

            @pl.loop(0, ROWS_PER_GROUP * D // L, unroll=8)
            def _(i):
                acc_v[pl.ds(i * L, L)] = zero16

            a0 = start - (start & 7)
            nsw = (end - a0 + WBUF - 1) // WBUF

            @pl.loop(0, nsw)
            def _(w):
                k0g = pl.multiple_of(a0 + w * WBUF, 8)
                pltpu.sync_copy(rows_hbm.at[pl.ds(k0g, WBUF)], rows_buf)
                pltpu.sync_copy(cols_hbm.at[pl.ds(k0g, WBUF)], cols_buf)
                pltpu.sync_copy(vals_hbm.at[pl.ds(k0g, WBUF)], vals_buf)
                wend = jnp.minimum(end - k0g, WBUF)
                nblk = (wend + W - 1) // W

                def issue(t, b):
                    tb = jnp.minimum(t, nblk - 1)
                    src = w_hbm.at[cols_buf.at[pl.ds(tb * W, W)]]
                    pltpu.async_copy(src, g[b], sems[b])

                def wait(b):
                    pltpu.make_async_copy(
                        w_hbm.at[cols_buf.at[pl.ds(0, W)]], g[b],
                        sems[b]).wait()

                def compute(t, b):
                    tb = jnp.minimum(t, nblk - 1)
                    live = t < nblk
                    for g16 in range(W // L):
                        kbase = tb * W + g16 * L
                        rv = rows_buf[pl.ds(kbase, L)]
                        vv = vals_buf[pl.ds(kbase, L)]
                        pos = (k0g + kbase) + lane
                        valid = (pos >= start) & (pos < end) & live
                        v_eff = jnp.where(valid, vv, 0.0)
                        lr = jnp.clip(rv - base_row, 0, ROWS_PER_GROUP - 1)

                        @plsc.parallel_loop(0, L, unroll=2)
                        def _(j):
                            jf = jnp.zeros((L,), jnp.int32) + j
                            v_j = _bcast_lane(v_eff, jf)
                            lr_j = _bcast_lane(lr, jf)
                            gf = jf + g16 * L
                            abase = lr_j * D + lane
                            for dj in range(D // L):
                                g16v = plsc.load_gather(g[b], [gf, idx1[dj]])
                                plsc.addupdate_scatter(
                                    acc_v, [abase + dj * L], v_j * g16v)

                for b in range(NBUF):
                    issue(b, b)

                @pl.loop(0, (nblk + NBUF - 1) // NBUF)
                def _(u):
                    for b in range(NBUF):
                        t = u * NBUF + b
                        wait(b)
                        compute(t, b)
                        issue(t + NBUF, b)

                for b in range(NBUF):
                    wait(b)

            pltpu.sync_copy(
                acc_v, out_hbm.at[pl.ds(base_row * D, ROWS_PER_GROUP * D)])

    return sc_kernel(rows_p, cols_p, vals_p, offs, weight)


def kernel(inp_rows, inp_cols, inp_values, weight):
    offs = jnp.searchsorted(
        inp_rows, jnp.arange(0, N + 1, ROWS_PER_GROUP), side="left"
    ).astype(jnp.int32)
    offs = jnp.pad(offs, (0, OFFS_PAD - offs.shape[0]), constant_values=NNZ)
    pad = NNZ_PAD - NNZ
    rows_p = jnp.pad(inp_rows, (0, pad), constant_values=N - 1)
    cols_p = jnp.pad(inp_cols, (0, pad), constant_values=0)
    vals_p = jnp.pad(inp_values, (0, pad), constant_values=0.0)
    out_flat = _sc_spmm(rows_p, cols_p, vals_p, offs, weight)
    return out_flat.reshape(N, D)
